# baseline (device time: 79855 ns/iter reference)
import jax
import jax.numpy as jnp
from jax import lax
from jax.experimental import pallas as pl
from jax.experimental.pallas import tpu as pltpu

N_DEV = 8
M = 2048
N = 2048

GROUPS = ((0, 768), (768, 640), (1408, 640))

GMASKS = ((1, 3, 4), (3, 4, 1), (4, 1, 3))


def kernel(A, B):
    def body(a_ref, b_ref, out_ref, b16_ref, *scratch):
        rbufs = [scratch[3 * g : 3 * g + 3] for g in range(3)]
        rs_ssem, rs_rsem, ag_ssem, ag_rsem = scratch[9:]

        my = lax.axis_index("i")

        def side_bits(q):
            qb0 = q & 1
            qb1 = (q >> 1) & 1
            qb2 = (q >> 2) & 1
            return (
                (qb0 ^ qb1, qb0, qb2),
                (qb1, qb2, qb0),
                (qb2, qb0 ^ qb1, qb0),
            )

        gsides = side_bits(my)

        b16_ref[:, :] = b_ref[:, :].astype(jnp.bfloat16)

        def owned_base(g, q):
            off, glen = GROUPS[g]
            s = side_bits(q)[g]
            return off + s[0] * (glen >> 1) + s[1] * (glen >> 2) + s[2] * (glen >> 3)

        barrier_sem = pltpu.get_barrier_semaphore()
        for m_ in (1, 3, 4):
            pl.semaphore_signal(
                barrier_sem,
                inc=1,
                device_id=(my ^ m_,),
                device_id_type=pl.DeviceIdType.MESH,
            )
        pl.semaphore_wait(barrier_sem, 3)

        bases = [None, None, None]
        pend = [None, None, None]
        done = []

        def rs_idx(g, t, h):
            return 5 * g + 2 * t + h

        def start_rs(g, t):
            glen = GROUPS[g][1]
            length = glen >> (t + 1)
            side = gsides[g][t]
            keep = bases[g] + side * length
            send = bases[g] + (1 - side) * length
            partner = my ^ GMASKS[g][t]
            if t < 2:
                half = length // 2
                side_n = side_bits(partner)[g][t + 1]
                rdmas = []
                for h, x in enumerate(((1 - side_n) * half, side_n * half)):
                    rdma = pltpu.make_async_remote_copy(
                        src_ref=out_ref.at[pl.ds(send + x, half), :],
                        dst_ref=rbufs[g][t].at[pl.ds(x, half), :],
                        send_sem=rs_ssem.at[rs_idx(g, t, h)],
                        recv_sem=rs_rsem.at[rs_idx(g, t, h)],
                        device_id=(partner,),
                        device_id_type=pl.DeviceIdType.MESH,
                    )
                    rdma.start()
                    rdmas.append(rdma)
                pend[g] = (rdmas, keep)
            else:
                rdma = pltpu.make_async_remote_copy(
                    src_ref=out_ref.at[pl.ds(send, length), :],
                    dst_ref=rbufs[g][t],
                    send_sem=rs_ssem.at[rs_idx(g, t, 0)],
                    recv_sem=rs_rsem.at[rs_idx(g, t, 0)],
                    device_id=(partner,),
                    device_id_type=pl.DeviceIdType.MESH,
                )
                rdma.start()
                pend[g] = ([rdma], keep)

        def ag_send(g, piece_origin, to_partner, slot):
            width = GROUPS[g][1] >> 3
            row = owned_base(g, piece_origin)
            rdma = pltpu.make_async_remote_copy(
                src_ref=out_ref.at[pl.ds(row, width), :],
                dst_ref=out_ref.at[pl.ds(row, width), :],
                send_sem=ag_ssem.at[7 * g + slot],
                recv_sem=ag_rsem.at[7 * g + slot],
                device_id=(to_partner,),
                device_id_type=pl.DeviceIdType.MESH,
            )
            rdma.start()
            done.append(rdma)

        def ag_wait(g, slot):
            width = GROUPS[g][1] >> 3
            rdma = pltpu.make_async_remote_copy(
                src_ref=out_ref.at[pl.ds(0, width), :],
                dst_ref=out_ref.at[pl.ds(0, width), :],
                send_sem=ag_ssem.at[7 * g + slot],
                recv_sem=ag_rsem.at[7 * g + slot],
                device_id=(my,),
                device_id_type=pl.DeviceIdType.MESH,
            )
            rdma.wait_recv()

        def dot_rows(start, length):
            out_ref[pl.ds(start, length), :] = jnp.dot(
                a_ref[pl.ds(start, length), :].astype(jnp.bfloat16),
                b16_ref[:, :],
                preferred_element_type=jnp.float32,
            ).astype(jnp.bfloat16)

        for g, (off, glen) in enumerate(GROUPS):
            length = glen // 2
            side = gsides[g][0]
            send = off + (1 - side) * length
            keep = off + side * length
            partner = my ^ GMASKS[g][0]
            half = length // 2
            side_n = side_bits(partner)[g][1]
            rdmas = []
            for h, x in enumerate(((1 - side_n) * half, side_n * half)):
                dot_rows(send + x, half)
                rdma = pltpu.make_async_remote_copy(
                    src_ref=out_ref.at[pl.ds(send + x, half), :],
                    dst_ref=rbufs[g][0].at[pl.ds(x, half), :],
                    send_sem=rs_ssem.at[rs_idx(g, 0, h)],
                    recv_sem=rs_rsem.at[rs_idx(g, 0, h)],
                    device_id=(partner,),
                    device_id_type=pl.DeviceIdType.MESH,
                )
                rdma.start()
                rdmas.append(rdma)
            pend[g] = (rdmas, keep)
            dot_rows(keep, length)

        for t in range(3):
            for g in range(3):
                rdmas, keep = pend[g]
                length = GROUPS[g][1] >> (t + 1)
                bases[g] = keep
                if t < 2:
                    half = length // 2
                    side_n = gsides[g][t + 1]
                    xs = ((1 - side_n) * half, side_n * half)
                    for h, x in enumerate(xs):
                        rdmas[h].wait_recv()
                        done.append(rdmas[h])
                        out_ref[pl.ds(keep + x, half), :] = (
                            out_ref[pl.ds(keep + x, half), :]
                            + rbufs[g][t][pl.ds(x, half), :]
                        )
                        if h == 0:
                            start_rs(g, t + 1)
                else:
                    rdmas[0].wait_recv()
                    done.append(rdmas[0])
                    out_ref[pl.ds(keep, length), :] = (
                        out_ref[pl.ds(keep, length), :] + rbufs[g][t][:, :]
                    )
                    m0, m1, m2 = (GMASKS[g][tt] for tt in (0, 1, 2))
                    ag_send(g, my, my ^ m2, 0)
                    ag_send(g, my, my ^ m1, 1)
                    ag_send(g, my, my ^ m0, 3)

        for g in range(3):
            m0, m1, m2 = (GMASKS[g][tt] for tt in (0, 1, 2))
            ag_wait(g, 0)
            ag_send(g, my ^ m2, my ^ m1, 2)
            ag_send(g, my ^ m2, my ^ m0, 4)
        for g in range(3):
            m0, m1, m2 = (GMASKS[g][tt] for tt in (0, 1, 2))
            ag_wait(g, 1)
            ag_send(g, my ^ m1, my ^ m0, 5)
            ag_wait(g, 2)
            ag_send(g, my ^ m1 ^ m2, my ^ m0, 6)
        for g in range(3):
            for slot in (3, 4, 5, 6):
                ag_wait(g, slot)

        for rdma in done:
            rdma.wait_send()

    scratch_shapes = [pltpu.VMEM((1024, N), jnp.bfloat16)]
    for _, glen in GROUPS:
        for t in range(3):
            scratch_shapes.append(
                pltpu.VMEM((glen >> (t + 1), N), jnp.bfloat16)
            )
    scratch_shapes += [
        pltpu.SemaphoreType.DMA((15,)),
        pltpu.SemaphoreType.DMA((15,)),
        pltpu.SemaphoreType.DMA((21,)),
        pltpu.SemaphoreType.DMA((21,)),
    ]

    return pl.pallas_call(
        body,
        out_shape=jax.ShapeDtypeStruct((M, N), jnp.bfloat16),
        in_specs=[
            pl.BlockSpec(memory_space=pltpu.VMEM),
            pl.BlockSpec(memory_space=pltpu.VMEM),
        ],
        out_specs=pl.BlockSpec(memory_space=pltpu.VMEM),
        scratch_shapes=scratch_shapes,
        compiler_params=pltpu.CompilerParams(collective_id=0),
    )(A, B)
